# Initial kernel scaffold; baseline (speedup 1.0000x reference)
import jax
import jax.numpy as jnp
from jax.experimental import pallas as pl


def kernel(input, weight):
    raise NotImplementedError("write your pallas kernel here")


# speedup vs baseline: 1.4952x; 1.4952x over previous
import functools

import jax
import jax.numpy as jnp
from jax import lax
from jax.experimental import pallas as pl
from jax.experimental.pallas import tpu as pltpu
from jax.experimental.pallas import tpu_sc as plsc

DIM = 32


@functools.cache
def _make_gather(total: int, chunk: int):
    nw = 32
    per_w = total // nw
    nchunk = per_w // chunk
    assert per_w % chunk == 0 and total % nw == 0

    mesh = plsc.VectorSubcoreMesh(core_axis_name="c", subcore_axis_name="s")

    @functools.partial(
        pl.kernel,
        mesh=mesh,
        out_type=jax.ShapeDtypeStruct((total, DIM), jnp.float32),
        scratch_types=[
            pltpu.VMEM((chunk,), jnp.int32),
            pltpu.VMEM((chunk, DIM), jnp.float32),
            pltpu.SemaphoreType.DMA,
        ],
        compiler_params=pltpu.CompilerParams(use_tc_tiling_on_sc=False),
    )
    def gather_kernel(table_hbm, idx_hbm, out_hbm, idx_v, rows_v, sem):
        wid = lax.axis_index("s") * 2 + lax.axis_index("c")
        base = wid * per_w

        def body(g, carry):
            off = base + g * chunk
            pltpu.sync_copy(idx_hbm.at[pl.ds(off, chunk)], idx_v)
            pltpu.async_copy(table_hbm.at[idx_v], rows_v, sem).wait()
            pltpu.sync_copy(rows_v, out_hbm.at[pl.ds(off, chunk)])
            return carry

        lax.fori_loop(0, nchunk, body, 0)

    return gather_kernel


def kernel(input, weight):
    idx = input.reshape(-1).astype(jnp.int32)
    out = _make_gather(idx.shape[0], 3200)(weight, idx)
    return out.reshape(input.shape + (DIM,))


# speedup vs baseline: 1.5009x; 1.0038x over previous
import functools

import jax
import jax.numpy as jnp
from jax import lax
from jax.experimental import pallas as pl
from jax.experimental.pallas import tpu as pltpu
from jax.experimental.pallas import tpu_sc as plsc

DIM = 32


@functools.cache
def _make_gather(total: int, chunk: int):
    nw = 32
    per_w = total // nw
    nchunk = per_w // chunk
    assert per_w % chunk == 0 and total % nw == 0

    mesh = plsc.VectorSubcoreMesh(core_axis_name="c", subcore_axis_name="s")

    @functools.partial(
        pl.kernel,
        mesh=mesh,
        out_type=jax.ShapeDtypeStruct((total, DIM), jnp.float32),
        scratch_types=[
            pltpu.VMEM((per_w,), jnp.int32),
            pltpu.VMEM((2, chunk, DIM), jnp.float32),
            pltpu.SemaphoreType.DMA,
            pltpu.SemaphoreType.DMA,
        ],
        compiler_params=pltpu.CompilerParams(use_tc_tiling_on_sc=False),
    )
    def gather_kernel(table_hbm, idx_hbm, out_hbm, idx_v, rows_v, gsem, ssem):
        wid = lax.axis_index("s") * 2 + lax.axis_index("c")
        base = wid * per_w

        pltpu.sync_copy(idx_hbm.at[pl.ds(base, per_w)], idx_v)

        def gather(g):
            return pltpu.async_copy(
                table_hbm.at[idx_v.at[pl.ds(g * chunk, chunk)]],
                rows_v.at[g % 2],
                gsem,
            )

        def store(g):
            return pltpu.async_copy(
                rows_v.at[g % 2],
                out_hbm.at[pl.ds(base + g * chunk, chunk)],
                ssem,
            )

        gathers = [None] * nchunk
        stores = [None] * nchunk
        gathers[0] = gather(0)
        for g in range(nchunk):
            if g + 1 < nchunk:
                if g >= 1:
                    stores[g - 1].wait()
                gathers[g + 1] = gather(g + 1)
            gathers[g].wait()
            stores[g] = store(g)
        if nchunk >= 2:
            stores[nchunk - 2].wait()
        stores[nchunk - 1].wait()

    return gather_kernel


def kernel(input, weight):
    idx = input.reshape(-1).astype(jnp.int32)
    out = _make_gather(idx.shape[0], 1600)(weight, idx)
    return out.reshape(input.shape + (DIM,))
